# baseline (speedup 1.0000x reference)
import functools

import numpy as np
import jax
import jax.numpy as jnp
from jax import lax
from jax.experimental import pallas as pl
from jax.experimental.pallas import tpu as pltpu
from jax.experimental.pallas import tpu_sc as plsc

N, J, D, H, W = 4, 17, 64, 64, 64
V = D * H * W
NJ = N * J
NUM_SAMPLES = 16
BV = 2048
G = V // BV

_KEYS_U32 = np.array([
    [1832780943, 270669613],
    [64467757, 2916123636],
    [2465931498, 255383827],
    [3134548294, 894150801],
    [2954079971, 3276725750],
    [2765691542, 824333390],
    [2768684296, 3055579793],
    [2547012911, 1371500959],
    [1016697191, 2390192106],
    [1128875147, 2463678267],
    [1039196627, 1683848162],
    [246739928, 3519402408],
    [3114009986, 1419417030],
    [3514951389, 229662949],
    [2526883203, 3973959769],
    [991576401, 3935454969],
], dtype=np.uint64).astype(np.uint32)
_KEYS_I32 = np.ascontiguousarray(_KEYS_U32).view(np.int32).reshape(-1)

_ROTS = (13, 15, 26, 6, 17, 29, 16, 24, 13, 15, 26, 6, 17, 29, 16, 24,
         13, 15, 26, 6)
_TINY = np.float32(np.finfo(np.float32).tiny)
_IMAX = np.int32(2**31 - 1)


def _srl(x, r):
    return lax.shift_right_logical(x, jnp.full(x.shape, r, jnp.int32))


def _rotl(x, r):
    return (x << r) | _srl(x, 32 - r)


def _threefry_bits(c1, k1, k2):
    ks2 = k1 ^ k2 ^ np.int32(0x1BD11BDA)
    ks = (k1, k2, ks2)
    x0 = jnp.full(c1.shape, 0, jnp.int32) + k1
    x1 = c1 + k2
    for i in range(5):
        for r in _ROTS[4 * i:4 * i + 4]:
            x0 = x0 + x1
            x1 = _rotl(x1, r)
            x1 = x1 ^ x0
        x0 = x0 + ks[(i + 1) % 3]
        x1 = x1 + ks[(i + 2) % 3] + np.int32(i + 1)
    return x0 ^ x1


def _sample_kernel(keys_ref, logits_ref, samples_ref, lse_ref,
                   accv_ref, acci_ref, acce_ref):
    i = pl.program_id(0)

    @pl.when(i == 0)
    def _init():
        accv_ref[...] = jnp.full((NUM_SAMPLES, NJ, BV), -jnp.inf, jnp.float32)
        acci_ref[...] = jnp.zeros((NUM_SAMPLES, NJ, BV), jnp.int32)
        acce_ref[...] = jnp.zeros((NJ, BV), jnp.float32)

    logits_blk = logits_ref[...]
    row_iota = lax.broadcasted_iota(jnp.int32, (NJ, BV), 0)
    col_iota = lax.broadcasted_iota(jnp.int32, (NJ, BV), 1)
    vbase = i * BV
    c1 = row_iota * V + (col_iota + vbase)
    idx_mat = col_iota + vbase

    acce_ref[...] = acce_ref[...] + jnp.exp(logits_blk)

    def body(s, _):
        k1 = keys_ref[2 * s]
        k2 = keys_ref[2 * s + 1]
        bits = _threefry_bits(c1, k1, k2)
        fb = _srl(bits, 9) | np.int32(0x3F800000)
        f = lax.bitcast_convert_type(fb, jnp.float32) - np.float32(1.0)
        u = jnp.maximum(f, _TINY)
        t = logits_blk - jnp.log(-jnp.log(u))
        av = accv_ref[s]
        upd = t > av
        accv_ref[s] = jnp.where(upd, t, av)
        acci_ref[s] = jnp.where(upd, idx_mat, acci_ref[s])
        return 0

    lax.fori_loop(0, NUM_SAMPLES, body, 0, unroll=2)

    @pl.when(i == G - 1)
    def _finalize():
        for s in range(NUM_SAMPLES):
            av = accv_ref[s]
            m = jnp.max(av, axis=1, keepdims=True)
            cand = jnp.where(av == m, acci_ref[s], _IMAX)
            samples_ref[:, s:s + 1] = jnp.min(cand, axis=1, keepdims=True)
        tot = jnp.sum(acce_ref[...], axis=1, keepdims=True)
        lse_ref[...] = jnp.broadcast_to(jnp.log(tot), (NJ, NUM_SAMPLES))


def _run_sampler(logits2d):
    keys = jnp.asarray(_KEYS_I32)
    grid_spec = pltpu.PrefetchScalarGridSpec(
        num_scalar_prefetch=1,
        grid=(G,),
        in_specs=[pl.BlockSpec((NJ, BV), lambda i, keys: (0, i))],
        out_specs=[
            pl.BlockSpec((NJ, NUM_SAMPLES), lambda i, keys: (0, 0)),
            pl.BlockSpec((NJ, NUM_SAMPLES), lambda i, keys: (0, 0)),
        ],
        scratch_shapes=[
            pltpu.VMEM((NUM_SAMPLES, NJ, BV), jnp.float32),
            pltpu.VMEM((NUM_SAMPLES, NJ, BV), jnp.int32),
            pltpu.VMEM((NJ, BV), jnp.float32),
        ],
    )
    samples_t, lse_b = pl.pallas_call(
        _sample_kernel,
        grid_spec=grid_spec,
        out_shape=[
            jax.ShapeDtypeStruct((NJ, NUM_SAMPLES), jnp.int32),
            jax.ShapeDtypeStruct((NJ, NUM_SAMPLES), jnp.float32),
        ],
    )(keys, logits2d)
    return samples_t, lse_b


_SC_MESH = plsc.VectorSubcoreMesh(core_axis_name="c", subcore_axis_name="s")


@functools.partial(
    pl.kernel,
    mesh=_SC_MESH,
    out_type=jax.ShapeDtypeStruct((16,), jnp.float32),
    scratch_types=[
        pltpu.VMEM((NJ, 16), jnp.int32),
        pltpu.VMEM((NJ, 16), jnp.float32),
        pltpu.VMEM((6 * NJ, 16), jnp.float32),
        pltpu.VMEM((NJ, 16), jnp.float32),
        pltpu.VMEM((16,), jnp.float32),
        pltpu.SemaphoreType.DMA,
        pltpu.SemaphoreType.DMA,
    ],
)
def _sc_tail(preds_hbm, samples_hbm, lse_hbm, gtv_hbm, out_hbm,
             svm, lvm, gvm, gat, ovm, sem, gsem):
    cid = lax.axis_index("c")
    sid = lax.axis_index("s")

    @pl.when((cid == 0) & (sid == 0))
    def _work():
        pltpu.async_copy(samples_hbm, svm, sem).wait()
        pltpu.async_copy(lse_hbm, lvm, sem).wait()
        pltpu.async_copy(gtv_hbm, gvm, sem).wait()
        copies = []
        for nj in range(NJ):
            idx = svm[nj] + np.int32(nj * V)
            copies.append(pltpu.async_copy(preds_hbm.at[idx], gat.at[nj], gsem))
        for c in copies:
            c.wait()
        inv64 = np.float32(1.0 / 64.0)
        half = np.float32(0.5)
        tot_vec = jnp.zeros((16,), jnp.float32)
        for n in range(N):
            dsum = jnp.zeros((16,), jnp.float32)
            rsum = jnp.zeros((16,), jnp.float32)
            for j in range(J):
                nj = n * J + j
                v = svm[nj]
                g = gat[nj]
                lp = g - lvm[nj]
                rsum = rsum + np.float32(1.0) / (-lp)
                x = (v & 63).astype(jnp.float32) * inv64 - half
                y = ((v >> 6) & 63).astype(jnp.float32) * inv64 - half
                z = (v >> 12).astype(jnp.float32) * inv64 - half
                d = (jnp.abs(x - gvm[nj]) * gvm[3 * NJ + nj]
                     + jnp.abs(y - gvm[NJ + nj]) * gvm[4 * NJ + nj]
                     + jnp.abs(z - gvm[2 * NJ + nj]) * gvm[5 * NJ + nj])
                dsum = dsum + d
            tot_vec = tot_vec + dsum * rsum
        ovm[...] = tot_vec * np.float32(1.0 / (N * J * NUM_SAMPLES))
        pltpu.async_copy(ovm, out_hbm, sem).wait()


def kernel(preds, batch_joints, batch_joints_vis):
    logits2d = preds.reshape(NJ, V)
    samples_t, lse_b = _run_sampler(logits2d)

    gt = batch_joints.reshape(NJ, 3).astype(jnp.float32)
    vis = batch_joints_vis.reshape(NJ, 3).astype(jnp.float32)
    gtv = jnp.concatenate(
        [gt[:, 0], gt[:, 1], gt[:, 2], vis[:, 0], vis[:, 1], vis[:, 2]], axis=0
    )
    gtv_b = jnp.broadcast_to(gtv[:, None], (6 * NJ, 16))

    preds_flat = preds.reshape(-1)
    out_vec = _sc_tail(preds_flat, samples_t, lse_b, gtv_b)
    return out_vec.sum()


# speedup vs baseline: 1.0074x; 1.0074x over previous
import functools

import numpy as np
import jax
import jax.numpy as jnp
from jax import lax
from jax.experimental import pallas as pl
from jax.experimental.pallas import tpu as pltpu
from jax.experimental.pallas import tpu_sc as plsc

N, J, D, H, W = 4, 17, 64, 64, 64
V = D * H * W
NJ = N * J
NUM_SAMPLES = 16
BV = 4096
G = V // BV

_KEYS_U32 = np.array([
    [1832780943, 270669613],
    [64467757, 2916123636],
    [2465931498, 255383827],
    [3134548294, 894150801],
    [2954079971, 3276725750],
    [2765691542, 824333390],
    [2768684296, 3055579793],
    [2547012911, 1371500959],
    [1016697191, 2390192106],
    [1128875147, 2463678267],
    [1039196627, 1683848162],
    [246739928, 3519402408],
    [3114009986, 1419417030],
    [3514951389, 229662949],
    [2526883203, 3973959769],
    [991576401, 3935454969],
], dtype=np.uint64).astype(np.uint32)
_KEYS_I32 = np.ascontiguousarray(_KEYS_U32).view(np.int32).reshape(-1)

_ROTS = (13, 15, 26, 6, 17, 29, 16, 24, 13, 15, 26, 6, 17, 29, 16, 24,
         13, 15, 26, 6)
_TINY = np.float32(np.finfo(np.float32).tiny)
_IMAX = np.int32(2**31 - 1)


def _srl(x, r):
    return lax.shift_right_logical(x, jnp.full(x.shape, r, jnp.int32))


def _rotl(x, r):
    return (x << r) | _srl(x, 32 - r)


def _threefry_bits(c1, k1, k2):
    ks2 = k1 ^ k2 ^ np.int32(0x1BD11BDA)
    ks = (k1, k2, ks2)
    x1 = c1 + k2
    x0 = x1 + k1
    x1 = _rotl(x1, _ROTS[0])
    x1 = x1 ^ x0
    first = True
    for i in range(5):
        for r in _ROTS[4 * i:4 * i + 4]:
            if first:
                first = False
                continue
            x0 = x0 + x1
            x1 = _rotl(x1, r)
            x1 = x1 ^ x0
        x0 = x0 + ks[(i + 1) % 3]
        x1 = x1 + ks[(i + 2) % 3] + np.int32(i + 1)
    return x0 ^ x1


def _sample_kernel(keys_ref, logits_ref, samples_ref, lse_ref,
                   accv_ref, acci_ref, acce_ref):
    i = pl.program_id(0)

    @pl.when(i == 0)
    def _init():
        accv_ref[...] = jnp.full((NUM_SAMPLES, NJ, BV), -jnp.inf, jnp.float32)
        acci_ref[...] = jnp.zeros((NUM_SAMPLES, NJ, BV), jnp.int32)
        acce_ref[...] = jnp.zeros((NJ, BV), jnp.float32)

    logits_blk = logits_ref[...]
    row_iota = lax.broadcasted_iota(jnp.int32, (NJ, BV), 0)
    col_iota = lax.broadcasted_iota(jnp.int32, (NJ, BV), 1)
    vbase = i * BV
    c1 = row_iota * V + (col_iota + vbase)
    idx_mat = col_iota + vbase

    acce_ref[...] = acce_ref[...] + jnp.exp(logits_blk)

    def body(s, _):
        k1 = keys_ref[2 * s]
        k2 = keys_ref[2 * s + 1]
        bits = _threefry_bits(c1, k1, k2)
        fb = _srl(bits, 9) | np.int32(0x3F800000)
        f = lax.bitcast_convert_type(fb, jnp.float32) - np.float32(1.0)
        u = jnp.maximum(f, _TINY)
        t = logits_blk - jnp.log(-jnp.log(u))
        av = accv_ref[s]
        upd = t > av
        accv_ref[s] = jnp.maximum(t, av)
        acci_ref[s] = jnp.where(upd, idx_mat, acci_ref[s])
        return 0

    lax.fori_loop(0, NUM_SAMPLES, body, 0, unroll=False)

    @pl.when(i == G - 1)
    def _finalize():
        for s in range(NUM_SAMPLES):
            av = accv_ref[s]
            m = jnp.max(av, axis=1, keepdims=True)
            cand = jnp.where(av == m, acci_ref[s], _IMAX)
            samples_ref[:, s:s + 1] = jnp.min(cand, axis=1, keepdims=True)
        tot = jnp.sum(acce_ref[...], axis=1, keepdims=True)
        lse_ref[...] = jnp.broadcast_to(jnp.log(tot), (NJ, NUM_SAMPLES))


def _run_sampler(logits2d):
    keys = jnp.asarray(_KEYS_I32)
    grid_spec = pltpu.PrefetchScalarGridSpec(
        num_scalar_prefetch=1,
        grid=(G,),
        in_specs=[pl.BlockSpec((NJ, BV), lambda i, keys: (0, i))],
        out_specs=[
            pl.BlockSpec((NJ, NUM_SAMPLES), lambda i, keys: (0, 0)),
            pl.BlockSpec((NJ, NUM_SAMPLES), lambda i, keys: (0, 0)),
        ],
        scratch_shapes=[
            pltpu.VMEM((NUM_SAMPLES, NJ, BV), jnp.float32),
            pltpu.VMEM((NUM_SAMPLES, NJ, BV), jnp.int32),
            pltpu.VMEM((NJ, BV), jnp.float32),
        ],
    )
    samples_t, lse_b = pl.pallas_call(
        _sample_kernel,
        grid_spec=grid_spec,
        out_shape=[
            jax.ShapeDtypeStruct((NJ, NUM_SAMPLES), jnp.int32),
            jax.ShapeDtypeStruct((NJ, NUM_SAMPLES), jnp.float32),
        ],
    )(keys, logits2d)
    return samples_t, lse_b


_SC_MESH = plsc.VectorSubcoreMesh(core_axis_name="c", subcore_axis_name="s")


@functools.partial(
    pl.kernel,
    mesh=_SC_MESH,
    out_type=jax.ShapeDtypeStruct((16,), jnp.float32),
    scratch_types=[
        pltpu.VMEM((NJ, 16), jnp.int32),
        pltpu.VMEM((NJ, 16), jnp.float32),
        pltpu.VMEM((6 * NJ, 16), jnp.float32),
        pltpu.VMEM((NJ, 16), jnp.float32),
        pltpu.VMEM((16,), jnp.float32),
        pltpu.SemaphoreType.DMA,
        pltpu.SemaphoreType.DMA,
    ],
)
def _sc_tail(preds_hbm, samples_hbm, lse_hbm, gtv_hbm, out_hbm,
             svm, lvm, gvm, gat, ovm, sem, gsem):
    cid = lax.axis_index("c")
    sid = lax.axis_index("s")

    @pl.when((cid == 0) & (sid == 0))
    def _work():
        pltpu.async_copy(samples_hbm, svm, sem).wait()
        pltpu.async_copy(lse_hbm, lvm, sem).wait()
        pltpu.async_copy(gtv_hbm, gvm, sem).wait()
        copies = []
        for nj in range(NJ):
            idx = svm[nj] + np.int32(nj * V)
            copies.append(pltpu.async_copy(preds_hbm.at[idx], gat.at[nj], gsem))
        for c in copies:
            c.wait()
        inv64 = np.float32(1.0 / 64.0)
        half = np.float32(0.5)
        tot_vec = jnp.zeros((16,), jnp.float32)
        for n in range(N):
            dsum = jnp.zeros((16,), jnp.float32)
            rsum = jnp.zeros((16,), jnp.float32)
            for j in range(J):
                nj = n * J + j
                v = svm[nj]
                g = gat[nj]
                lp = g - lvm[nj]
                rsum = rsum + np.float32(1.0) / (-lp)
                x = (v & 63).astype(jnp.float32) * inv64 - half
                y = ((v >> 6) & 63).astype(jnp.float32) * inv64 - half
                z = (v >> 12).astype(jnp.float32) * inv64 - half
                d = (jnp.abs(x - gvm[nj]) * gvm[3 * NJ + nj]
                     + jnp.abs(y - gvm[NJ + nj]) * gvm[4 * NJ + nj]
                     + jnp.abs(z - gvm[2 * NJ + nj]) * gvm[5 * NJ + nj])
                dsum = dsum + d
            tot_vec = tot_vec + dsum * rsum
        ovm[...] = tot_vec * np.float32(1.0 / (N * J * NUM_SAMPLES))
        pltpu.async_copy(ovm, out_hbm, sem).wait()


def kernel(preds, batch_joints, batch_joints_vis):
    logits2d = preds.reshape(NJ, V)
    samples_t, lse_b = _run_sampler(logits2d)

    gt = batch_joints.reshape(NJ, 3).astype(jnp.float32)
    vis = batch_joints_vis.reshape(NJ, 3).astype(jnp.float32)
    gtv = jnp.concatenate(
        [gt[:, 0], gt[:, 1], gt[:, 2], vis[:, 0], vis[:, 1], vis[:, 2]], axis=0
    )
    gtv_b = jnp.broadcast_to(gtv[:, None], (6 * NJ, 16))

    preds_flat = preds.reshape(-1)
    out_vec = _sc_tail(preds_flat, samples_t, lse_b, gtv_b)
    return out_vec.sum()


# speedup vs baseline: 1.0518x; 1.0440x over previous
import functools

import numpy as np
import jax
import jax.numpy as jnp
from jax import lax
from jax.experimental import pallas as pl
from jax.experimental.pallas import tpu as pltpu
from jax.experimental.pallas import tpu_sc as plsc

N, J, D, H, W = 4, 17, 64, 64, 64
V = D * H * W
NJ = N * J
NUM_SAMPLES = 16
BH = 2048

_KEYS_U32 = np.array([
    [1832780943, 270669613],
    [64467757, 2916123636],
    [2465931498, 255383827],
    [3134548294, 894150801],
    [2954079971, 3276725750],
    [2765691542, 824333390],
    [2768684296, 3055579793],
    [2547012911, 1371500959],
    [1016697191, 2390192106],
    [1128875147, 2463678267],
    [1039196627, 1683848162],
    [246739928, 3519402408],
    [3114009986, 1419417030],
    [3514951389, 229662949],
    [2526883203, 3973959769],
    [991576401, 3935454969],
], dtype=np.uint64).astype(np.uint32)
_KEYS_I32 = np.ascontiguousarray(_KEYS_U32).view(np.int32).reshape(-1)

_ROTS = (13, 15, 26, 6, 17, 29, 16, 24, 13, 15, 26, 6, 17, 29, 16, 24,
         13, 15, 26, 6)
_TINY = np.float32(np.finfo(np.float32).tiny)
_IMAX = np.int32(2**31 - 1)


def _srl(x, r):
    return lax.shift_right_logical(x, jnp.full(x.shape, r, jnp.int32))


def _rotl(x, r):
    return (x << r) | _srl(x, 32 - r)


def _threefry_bits(c1, k1, k2):
    ks2 = k1 ^ k2 ^ np.int32(0x1BD11BDA)
    ks = (k1, k2, ks2)
    x1 = c1 + k2
    x0 = x1 + k1
    x1 = _rotl(x1, _ROTS[0])
    x1 = x1 ^ x0
    first = True
    for i in range(5):
        for r in _ROTS[4 * i:4 * i + 4]:
            if first:
                first = False
                continue
            x0 = x0 + x1
            x1 = _rotl(x1, r)
            x1 = x1 ^ x0
        x0 = x0 + ks[(i + 1) % 3]
        x1 = x1 + ks[(i + 2) % 3] + np.int32(i + 1)
    return x0 ^ x1


RR = 2 * NJ
VH = V // 2
G = VH // BH


def _sample_kernel(keys_ref, logits_ref, samples_ref, lse_ref,
                   accv_ref, acci_ref, acce_ref):
    i = pl.program_id(0)

    @pl.when(i == 0)
    def _init():
        accv_ref[...] = jnp.full((NUM_SAMPLES, RR, BH), -jnp.inf, jnp.float32)
        acci_ref[...] = jnp.zeros((NUM_SAMPLES, RR, BH), jnp.int32)
        acce_ref[...] = jnp.zeros((RR, BH), jnp.float32)

    logits_blk = logits_ref[...]
    row_iota = lax.broadcasted_iota(jnp.int32, (RR, BH), 0)
    col_iota = lax.broadcasted_iota(jnp.int32, (RR, BH), 1)
    vbase = i * BH
    idx_mat = (row_iota & 1) * VH + (col_iota + vbase)
    c1 = _srl(row_iota, 1) * V + idx_mat

    acce_ref[...] = acce_ref[...] + jnp.exp(logits_blk)

    def body(s, _):
        k1 = keys_ref[2 * s]
        k2 = keys_ref[2 * s + 1]
        bits = _threefry_bits(c1, k1, k2)
        fb = _srl(bits, 9) | np.int32(0x3F800000)
        f = lax.bitcast_convert_type(fb, jnp.float32) - np.float32(1.0)
        u = jnp.maximum(f, _TINY)
        t = logits_blk - jnp.log(-jnp.log(u))
        av = accv_ref[s]
        upd = t > av
        accv_ref[s] = jnp.maximum(t, av)
        acci_ref[s] = jnp.where(upd, idx_mat, acci_ref[s])
        return 0

    lax.fori_loop(0, NUM_SAMPLES, body, 0, unroll=False)

    @pl.when(i == G - 1)
    def _finalize():
        for s in range(NUM_SAMPLES):
            av = accv_ref[s]
            m = jnp.max(av, axis=1, keepdims=True)
            cand = jnp.where(av == m, acci_ref[s], _IMAX)
            idx = jnp.min(cand, axis=1, keepdims=True)
            m2 = jnp.reshape(m, (NJ, 2))
            c2 = jnp.reshape(idx, (NJ, 2))
            better = m2[:, 0:1] >= m2[:, 1:2]
            samples_ref[:, s:s + 1] = jnp.where(better, c2[:, 0:1], c2[:, 1:2])
        tot = jnp.sum(acce_ref[...], axis=1, keepdims=True)
        t2 = jnp.reshape(tot, (NJ, 2))
        lse = jnp.log(t2[:, 0:1] + t2[:, 1:2])
        lse_ref[...] = jnp.broadcast_to(lse, (NJ, NUM_SAMPLES))


def _run_sampler(logits2d):
    keys = jnp.asarray(_KEYS_I32)
    logits_h = logits2d.reshape(RR, VH)
    grid_spec = pltpu.PrefetchScalarGridSpec(
        num_scalar_prefetch=1,
        grid=(G,),
        in_specs=[pl.BlockSpec((RR, BH), lambda i, keys: (0, i))],
        out_specs=[
            pl.BlockSpec((NJ, NUM_SAMPLES), lambda i, keys: (0, 0)),
            pl.BlockSpec((NJ, NUM_SAMPLES), lambda i, keys: (0, 0)),
        ],
        scratch_shapes=[
            pltpu.VMEM((NUM_SAMPLES, RR, BH), jnp.float32),
            pltpu.VMEM((NUM_SAMPLES, RR, BH), jnp.int32),
            pltpu.VMEM((RR, BH), jnp.float32),
        ],
    )
    samples_t, lse_b = pl.pallas_call(
        _sample_kernel,
        grid_spec=grid_spec,
        out_shape=[
            jax.ShapeDtypeStruct((NJ, NUM_SAMPLES), jnp.int32),
            jax.ShapeDtypeStruct((NJ, NUM_SAMPLES), jnp.float32),
        ],
    )(keys, logits_h)
    return samples_t, lse_b


@functools.cache
def _build_sc_tail():
    mesh = plsc.VectorSubcoreMesh(core_axis_name="c", subcore_axis_name="s")
    return functools.partial(
        pl.kernel,
        mesh=mesh,
        out_type=jax.ShapeDtypeStruct((16,), jnp.float32),
        scratch_types=[
            pltpu.VMEM((NJ, 16), jnp.int32),
            pltpu.VMEM((NJ, 16), jnp.float32),
            pltpu.VMEM((6 * NJ, 16), jnp.float32),
            pltpu.VMEM((NJ, 16), jnp.float32),
            pltpu.VMEM((16,), jnp.float32),
            pltpu.SemaphoreType.DMA,
            pltpu.SemaphoreType.DMA,
        ],
    )(_sc_tail_body)


def _sc_tail_body(preds_hbm, samples_hbm, lse_hbm, gtv_hbm, out_hbm,
                  svm, lvm, gvm, gat, ovm, sem, gsem):
    cid = lax.axis_index("c")
    sid = lax.axis_index("s")

    @pl.when((cid == 0) & (sid == 0))
    def _work():
        pltpu.async_copy(samples_hbm, svm, sem).wait()
        pltpu.async_copy(lse_hbm, lvm, sem).wait()
        pltpu.async_copy(gtv_hbm, gvm, sem).wait()
        copies = []
        for nj in range(NJ):
            idx = svm[nj] + np.int32(nj * V)
            copies.append(pltpu.async_copy(preds_hbm.at[idx], gat.at[nj], gsem))
        for c in copies:
            c.wait()
        inv64 = np.float32(1.0 / 64.0)
        half = np.float32(0.5)
        tot_vec = jnp.zeros((16,), jnp.float32)
        for n in range(N):
            dsum = jnp.zeros((16,), jnp.float32)
            rsum = jnp.zeros((16,), jnp.float32)
            for j in range(J):
                nj = n * J + j
                v = svm[nj]
                g = gat[nj]
                lp = g - lvm[nj]
                rsum = rsum + np.float32(1.0) / (-lp)
                x = (v & 63).astype(jnp.float32) * inv64 - half
                y = ((v >> 6) & 63).astype(jnp.float32) * inv64 - half
                z = (v >> 12).astype(jnp.float32) * inv64 - half
                d = (jnp.abs(x - gvm[nj]) * gvm[3 * NJ + nj]
                     + jnp.abs(y - gvm[NJ + nj]) * gvm[4 * NJ + nj]
                     + jnp.abs(z - gvm[2 * NJ + nj]) * gvm[5 * NJ + nj])
                dsum = dsum + d
            tot_vec = tot_vec + dsum * rsum
        ovm[...] = tot_vec * np.float32(1.0 / (N * J * NUM_SAMPLES))
        pltpu.async_copy(ovm, out_hbm, sem).wait()


def kernel(preds, batch_joints, batch_joints_vis):
    logits2d = preds.reshape(NJ, V)
    samples_t, lse_b = _run_sampler(logits2d)

    gt = batch_joints.reshape(NJ, 3).astype(jnp.float32)
    vis = batch_joints_vis.reshape(NJ, 3).astype(jnp.float32)
    gtv = jnp.concatenate(
        [gt[:, 0], gt[:, 1], gt[:, 2], vis[:, 0], vis[:, 1], vis[:, 2]], axis=0
    )
    gtv_b = jnp.broadcast_to(gtv[:, None], (6 * NJ, 16))

    preds_flat = preds.reshape(-1)
    out_vec = _build_sc_tail()(preds_flat, samples_t, lse_b, gtv_b)
    return out_vec.sum()
